# Initial kernel scaffold; baseline (speedup 1.0000x reference)
import jax
import jax.numpy as jnp
from jax.experimental import pallas as pl


def kernel(x, edge_index, W1, b1, W2, b2):
    raise NotImplementedError("write your pallas kernel here")


# speedup vs baseline: 14.0084x; 14.0084x over previous
import functools

import jax
import jax.numpy as jnp
from jax import lax
from jax.experimental import pallas as pl
from jax.experimental.pallas import tpu as pltpu
from jax.experimental.pallas import tpu_sc as plsc

N = 10000
E = 320000
D_IN = 165
D_HID = 128

NC = 2
NS = 16
EPC = E // NC
EPW = EPC // NS
CH = 80
NCHUNK = EPW // CH

NPAD = 10240
RPT = NPAD // NS
MRPT = NPAD // NS
ZR = 128

_mesh = plsc.VectorSubcoreMesh(core_axis_name="c", subcore_axis_name="s")


def _zeros16():
    return jnp.zeros((16,), jnp.float32)


def _sc_degree(dst):
    @functools.partial(
        pl.kernel,
        mesh=_mesh,
        out_type=(
            jax.ShapeDtypeStruct((NPAD,), jnp.float32),
            jax.ShapeDtypeStruct((NPAD,), jnp.float32),
        ),
        scratch_types=[
            pltpu.VMEM((CH,), jnp.int32),
            pltpu.VMEM((CH,), jnp.float32),
            pltpu.VMEM((RPT,), jnp.float32),
            pltpu.VMEM_SHARED((NPAD,), jnp.float32),
        ],
    )
    def deg_kernel(dst_hbm, out_a, out_b, dst_v, ones_v, zbuf, acc):
        cid = lax.axis_index("c")
        sid = lax.axis_index("s")
        for i in range(CH // 16):
            ones_v[pl.ds(i * 16, 16)] = jnp.ones((16,), jnp.float32)
        for i in range(RPT // 16):
            zbuf[pl.ds(i * 16, 16)] = _zeros16()
        pltpu.sync_copy(zbuf, acc.at[pl.ds(sid * RPT, RPT)])
        plsc.subcore_barrier()

        base = cid * EPC + sid * EPW

        def body(j, carry):
            pltpu.sync_copy(dst_hbm.at[pl.ds(base + j * CH, CH)], dst_v)
            pltpu.sync_copy(ones_v, acc.at[dst_v], add=True)
            return carry

        lax.fori_loop(0, NCHUNK, body, 0)
        plsc.subcore_barrier()

        @pl.when(cid == 0)
        def _():
            pltpu.sync_copy(acc.at[pl.ds(sid * RPT, RPT)],
                            out_a.at[pl.ds(sid * RPT, RPT)])

        @pl.when(cid == 1)
        def _():
            pltpu.sync_copy(acc.at[pl.ds(sid * RPT, RPT)],
                            out_b.at[pl.ds(sid * RPT, RPT)])

    return deg_kernel(dst)


def _sc_agg_rows(hs, src, dst):
    @functools.partial(
        pl.kernel,
        mesh=_mesh,
        out_type=(
            jax.ShapeDtypeStruct((NPAD, D_HID), jnp.float32),
            jax.ShapeDtypeStruct((NPAD, D_HID), jnp.float32),
        ),
        scratch_types=[
            pltpu.VMEM((CH,), jnp.int32),
            pltpu.VMEM((CH,), jnp.int32),
            pltpu.VMEM((CH, D_HID), jnp.float32),
            pltpu.VMEM((ZR, D_HID), jnp.float32),
            pltpu.VMEM_SHARED((NPAD, D_HID), jnp.float32),
            pltpu.SemaphoreType.DMA,
        ],
    )
    def agg_kernel(hs_hbm, src_hbm, dst_hbm, out_a, out_b,
                   src_v, dst_v, rows_v, zbuf, acc, sem):
        cid = lax.axis_index("c")
        sid = lax.axis_index("s")

        def zb(i, carry):
            zbuf[i // 8, pl.ds((i % 8) * 16, 16)] = _zeros16()
            return carry

        lax.fori_loop(0, (ZR * D_HID) // 16, zb, 0)

        def zc(k, carry):
            pltpu.sync_copy(zbuf, acc.at[pl.ds(sid * MRPT + k * ZR, ZR)])
            return carry

        lax.fori_loop(0, MRPT // ZR, zc, 0)
        plsc.subcore_barrier()

        base = cid * EPC + sid * EPW

        def body(j, carry):
            eb = base + j * CH
            pltpu.sync_copy(src_hbm.at[pl.ds(eb, CH)], src_v)
            pltpu.sync_copy(dst_hbm.at[pl.ds(eb, CH)], dst_v)
            pltpu.async_copy(hs_hbm.at[src_v], rows_v, sem).wait()
            pltpu.sync_copy(rows_v, acc.at[dst_v], add=True)
            return carry

        lax.fori_loop(0, NCHUNK, body, 0)
        plsc.subcore_barrier()

        @pl.when(cid == 0)
        def _():
            pltpu.sync_copy(acc.at[pl.ds(sid * MRPT, MRPT)],
                            out_a.at[pl.ds(sid * MRPT, MRPT)])

        @pl.when(cid == 1)
        def _():
            pltpu.sync_copy(acc.at[pl.ds(sid * MRPT, MRPT)],
                            out_b.at[pl.ds(sid * MRPT, MRPT)])

    return agg_kernel(hs, src, dst)


def _sc_agg_scalar(vs, src, dst):
    @functools.partial(
        pl.kernel,
        mesh=_mesh,
        out_type=(
            jax.ShapeDtypeStruct((NPAD,), jnp.float32),
            jax.ShapeDtypeStruct((NPAD,), jnp.float32),
        ),
        scratch_types=[
            pltpu.VMEM((CH,), jnp.int32),
            pltpu.VMEM((CH,), jnp.int32),
            pltpu.VMEM((CH,), jnp.float32),
            pltpu.VMEM((RPT,), jnp.float32),
            pltpu.VMEM_SHARED((NPAD,), jnp.float32),
            pltpu.SemaphoreType.DMA,
        ],
    )
    def aggs_kernel(vs_hbm, src_hbm, dst_hbm, out_a, out_b,
                    src_v, dst_v, vals_v, zbuf, acc, sem):
        cid = lax.axis_index("c")
        sid = lax.axis_index("s")
        for i in range(RPT // 16):
            zbuf[pl.ds(i * 16, 16)] = _zeros16()
        pltpu.sync_copy(zbuf, acc.at[pl.ds(sid * RPT, RPT)])
        plsc.subcore_barrier()

        base = cid * EPC + sid * EPW

        def body(j, carry):
            eb = base + j * CH
            pltpu.sync_copy(src_hbm.at[pl.ds(eb, CH)], src_v)
            pltpu.sync_copy(dst_hbm.at[pl.ds(eb, CH)], dst_v)
            pltpu.async_copy(vs_hbm.at[src_v], vals_v, sem).wait()
            pltpu.sync_copy(vals_v, acc.at[dst_v], add=True)
            return carry

        lax.fori_loop(0, NCHUNK, body, 0)
        plsc.subcore_barrier()

        @pl.when(cid == 0)
        def _():
            pltpu.sync_copy(acc.at[pl.ds(sid * RPT, RPT)],
                            out_a.at[pl.ds(sid * RPT, RPT)])

        @pl.when(cid == 1)
        def _():
            pltpu.sync_copy(acc.at[pl.ds(sid * RPT, RPT)],
                            out_b.at[pl.ds(sid * RPT, RPT)])

    return aggs_kernel(vs, src, dst)


BN = 1000


def _tc_mm1_body(x_ref, w_ref, da_ref, db_ref, hs_ref):
    dinv = lax.rsqrt(da_ref[...] + db_ref[...] + 1.0)
    h = jnp.dot(x_ref[...], w_ref[...], preferred_element_type=jnp.float32)
    hs_ref[...] = h * dinv


def _tc_mm1(x, w1, da, db):
    return pl.pallas_call(
        _tc_mm1_body,
        grid=(N // BN,),
        in_specs=[
            pl.BlockSpec((BN, D_IN), lambda i: (i, 0)),
            pl.BlockSpec((D_IN, D_HID), lambda i: (0, 0)),
            pl.BlockSpec((BN, 1), lambda i: (i, 0)),
            pl.BlockSpec((BN, 1), lambda i: (i, 0)),
        ],
        out_specs=pl.BlockSpec((BN, D_HID), lambda i: (i, 0)),
        out_shape=jax.ShapeDtypeStruct((N, D_HID), jnp.float32),
    )(x, w1, da, db)


def _tc_mm2_body(aa_ref, ab_ref, hs_ref, da_ref, db_ref, b1_ref, w2t_ref,
                 vs_ref):
    dinv = lax.rsqrt(da_ref[...] + db_ref[...] + 1.0)
    pre = dinv * (aa_ref[...] + ab_ref[...] + hs_ref[...]) + b1_ref[...]
    h = jnp.maximum(pre, 0.0)
    z = jnp.sum(h * w2t_ref[...], axis=1, keepdims=True)
    vs_ref[...] = dinv * z


def _tc_mm2(aa, ab, hs, da, db, b1r, w2t):
    return pl.pallas_call(
        _tc_mm2_body,
        grid=(N // BN,),
        in_specs=[
            pl.BlockSpec((BN, D_HID), lambda i: (i, 0)),
            pl.BlockSpec((BN, D_HID), lambda i: (i, 0)),
            pl.BlockSpec((BN, D_HID), lambda i: (i, 0)),
            pl.BlockSpec((BN, 1), lambda i: (i, 0)),
            pl.BlockSpec((BN, 1), lambda i: (i, 0)),
            pl.BlockSpec((1, D_HID), lambda i: (0, 0)),
            pl.BlockSpec((1, D_HID), lambda i: (0, 0)),
        ],
        out_specs=pl.BlockSpec((BN, 1), lambda i: (i, 0)),
        out_shape=jax.ShapeDtypeStruct((N, 1), jnp.float32),
    )(aa, ab, hs, da, db, b1r, w2t)


def _tc_fin_body(va_ref, vb_ref, vs_ref, da_ref, db_ref, b2_ref, out_ref):
    dinv = lax.rsqrt(da_ref[...] + db_ref[...] + 1.0)
    z = dinv * (va_ref[...] + vb_ref[...] + vs_ref[...]) + b2_ref[0, 0]
    out_ref[...] = jax.nn.sigmoid(z)


def _tc_fin(va, vb, vs, da, db, b2):
    shp = (80, 125)
    args = [a.reshape(shp) for a in (va, vb, vs, da, db)]
    out = pl.pallas_call(
        _tc_fin_body,
        in_specs=[pl.BlockSpec(shp, lambda: (0, 0))] * 5
        + [pl.BlockSpec((1, 1), lambda: (0, 0))],
        out_specs=pl.BlockSpec(shp, lambda: (0, 0)),
        out_shape=jax.ShapeDtypeStruct(shp, jnp.float32),
    )(*args, b2.reshape(1, 1))
    return out.reshape(N, 1)


def kernel(x, edge_index, W1, b1, W2, b2):
    src = edge_index[0]
    dst = edge_index[1]

    deg_a, deg_b = _sc_degree(dst)
    da = deg_a[:N].reshape(N, 1)
    db = deg_b[:N].reshape(N, 1)

    hs = _tc_mm1(x, W1, da, db)
    acc_a, acc_b = _sc_agg_rows(hs, src, dst)

    vs = _tc_mm2(acc_a, acc_b, hs, da, db,
                 b1.reshape(1, D_HID), W2.reshape(1, D_HID))
    vsf = vs.reshape(N)

    va, vb = _sc_agg_scalar(vsf, src, dst)
    out = _tc_fin(va[:N], vb[:N], vsf, da.reshape(N), db.reshape(N), b2)
    return out
